# baseline (speedup 1.0000x reference)
import functools

import jax
import jax.numpy as jnp
from jax import lax
from jax.experimental import pallas as pl
from jax.experimental.pallas import tpu as pltpu
from jax.experimental.pallas import tpu_sc as plsc

N = 16384
L = 16
D = 4
G = L * D
NB = 64
NC = 2
NS = 16
NW = NC * NS
ROWS = N // NW
LANES = 16
NJB = G // LANES

_mesh = plsc.VectorSubcoreMesh(
    core_axis_name="c", subcore_axis_name="s", num_cores=NC, num_subcores=NS)
_params = pltpu.CompilerParams(
    needs_layout_passes=False, use_tc_tiling_on_sc=False)


def _tc_params_body(x_ref, p_ref):
    x = x_ref[...]
    mn = jnp.min(x, axis=0)
    mx = jnp.max(x, axis=0)
    degen = jnp.abs(mx - mn) < 1e-10
    mx = jnp.where(degen, mx + 1e-05, mx)
    mn = jnp.where(degen, mn - 1e-05, mn)
    delta = (mx - mn) / NB
    z = jnp.zeros((G,), jnp.float32)
    p_ref[...] = jnp.stack(
        [mn, delta, 1.0 / delta, delta * 0.5, z, z, z, z])


_tc_params = pl.pallas_call(
    _tc_params_body,
    out_shape=jax.ShapeDtypeStruct((8, G), jnp.float32),
)


@functools.partial(
    pl.kernel,
    out_type=jax.ShapeDtypeStruct((NW, 2, G * NB), jnp.float32),
    mesh=_mesh,
    compiler_params=_params,
    scratch_types=[pltpu.VMEM((ROWS, G), jnp.float32),
                   pltpu.VMEM((ROWS, G), jnp.float32),
                   pltpu.VMEM((8, G), jnp.float32),
                   pltpu.VMEM((G * NB,), jnp.float32),
                   pltpu.VMEM((G * NB,), jnp.float32),
                   pltpu.SemaphoreType.DMA,
                   pltpu.SemaphoreType.DMA],
)
def _sc_hist(xr_hbm, xf_hbm, p_hbm, counts_hbm,
             rbuf, fbuf, pbuf, cr, cf, rsem, fsem):
    wid = lax.axis_index("s") * NC + lax.axis_index("c")
    rcp = pltpu.async_copy(xr_hbm.at[pl.ds(wid * ROWS, ROWS)], rbuf, rsem)
    fcp = pltpu.async_copy(xf_hbm.at[pl.ds(wid * ROWS, ROWS)], fbuf, fsem)
    pltpu.sync_copy(p_hbm, pbuf)

    zeros = jnp.zeros((LANES,), jnp.float32)

    @plsc.parallel_loop(0, G * NB // LANES, unroll=8)
    def _(i):
        cr[pl.ds(i * LANES, LANES)] = zeros
        cf[pl.ds(i * LANES, LANES)] = zeros

    mnb, deltab, invdb, halfwb, baseb = [], [], [], [], []
    for jb in range(NJB):
        sl = pl.ds(jb * LANES, LANES)
        mnb.append(pbuf[0, sl])
        deltab.append(pbuf[1, sl])
        invdb.append(pbuf[2, sl])
        halfwb.append(pbuf[3, sl])
        baseb.append((jnp.arange(LANES, dtype=jnp.int32) + jb * LANES) * NB)

    ones = jnp.ones((LANES,), jnp.float32)
    rcp.wait()
    fcp.wait()

    @plsc.parallel_loop(0, ROWS, unroll=8)
    def _(i):
        for jb in range(NJB):
            sl = pl.ds(jb * LANES, LANES)
            xr_v = rbuf[i, sl]
            tr = (xr_v - mnb[jb]) * invdb[jb]
            ir = tr.astype(jnp.int32)
            ir = jnp.minimum(jnp.maximum(ir, 0), NB - 1)
            plsc.addupdate_scatter(cr, [baseb[jb] + ir], ones)
            xf_v = fbuf[i, sl]
            tf = (xf_v - mnb[jb]) * invdb[jb]
            tf = jnp.minimum(jnp.maximum(tf, -1.0), 64.0)
            jf = tf.astype(jnp.int32)
            jf = jnp.minimum(jnp.maximum(jf, 0), NB - 1)
            center = mnb[jb] + deltab[jb] * (jf.astype(jnp.float32) + 0.5)
            hit = (halfwb[jb] - jnp.abs(xf_v - center)) > 0.0
            plsc.addupdate_scatter(cf, [baseb[jb] + jf], ones, mask=hit)

    pltpu.sync_copy(cr, counts_hbm.at[wid, 0])
    pltpu.sync_copy(cf, counts_hbm.at[wid, 1])


def _tc_finalize_body(c_ref, p_ref, o_ref):
    c = c_ref[...]
    tot = jnp.sum(c, axis=0)
    s = jnp.sum(jnp.abs(tot[1] - tot[0]), axis=1)
    delta = p_ref[1, :]
    o_ref[...] = s / (delta * float(NB * N))


_tc_finalize = pl.pallas_call(
    _tc_finalize_body,
    out_shape=jax.ShapeDtypeStruct((G,), jnp.float32),
)


def kernel(x_fake, x_real):
    xr = x_real.reshape(N, G)
    xf = x_fake.reshape(N, G)
    params = jnp.ones((8, G), jnp.float32)
    counts = _sc_hist(xr, xf, params)
    return counts[0, 0, :64].reshape(L, D)


# speedup vs baseline: 1.0624x; 1.0624x over previous
import functools

import jax
import jax.numpy as jnp
from jax import lax
from jax.experimental import pallas as pl
from jax.experimental.pallas import tpu as pltpu
from jax.experimental.pallas import tpu_sc as plsc

N = 16384
L = 16
D = 4
G = L * D
NB = 64
NC = 2
NS = 16
NW = NC * NS
ROWS = N // NW
LANES = 16
NJB = G // LANES
GS = NB + 1
CWORDS = G * GS

_mesh = plsc.VectorSubcoreMesh(
    core_axis_name="c", subcore_axis_name="s", num_cores=NC, num_subcores=NS)
_params = pltpu.CompilerParams(
    needs_layout_passes=False, use_tc_tiling_on_sc=False)


def _tc_params_body(x_ref, p_ref):
    x = x_ref[...]
    mn = jnp.min(x, axis=0)
    mx = jnp.max(x, axis=0)
    degen = jnp.abs(mx - mn) < 1e-10
    mx = jnp.where(degen, mx + 1e-05, mx)
    mn = jnp.where(degen, mn - 1e-05, mn)
    delta = (mx - mn) / NB
    z = jnp.zeros((G,), jnp.float32)
    p_ref[...] = jnp.stack(
        [mn, delta, 1.0 / delta, delta * 0.5, z, z, z, z])


_tc_params = pl.pallas_call(
    _tc_params_body,
    out_shape=jax.ShapeDtypeStruct((8, G), jnp.float32),
)


@functools.partial(
    pl.kernel,
    out_type=jax.ShapeDtypeStruct((NW, 2, CWORDS), jnp.float32),
    mesh=_mesh,
    compiler_params=_params,
    scratch_types=[pltpu.VMEM((ROWS, G), jnp.float32),
                   pltpu.VMEM((ROWS, G), jnp.float32),
                   pltpu.VMEM((8, G), jnp.float32),
                   pltpu.VMEM((CWORDS,), jnp.float32),
                   pltpu.VMEM((CWORDS,), jnp.float32),
                   pltpu.SemaphoreType.DMA,
                   pltpu.SemaphoreType.DMA],
)
def _sc_hist(xr_hbm, xf_hbm, p_hbm, counts_hbm,
             rbuf, fbuf, pbuf, cr, cf, rsem, fsem):
    wid = lax.axis_index("s") * NC + lax.axis_index("c")
    rcp = pltpu.async_copy(xr_hbm.at[pl.ds(wid * ROWS, ROWS)], rbuf, rsem)
    fcp = pltpu.async_copy(xf_hbm.at[pl.ds(wid * ROWS, ROWS)], fbuf, fsem)
    pltpu.sync_copy(p_hbm, pbuf)

    zeros = jnp.zeros((LANES,), jnp.float32)

    @plsc.parallel_loop(0, CWORDS // LANES, unroll=8)
    def _(i):
        cr[pl.ds(i * LANES, LANES)] = zeros
        cf[pl.ds(i * LANES, LANES)] = zeros

    mnb, deltab, invdb, halfwb, baseb = [], [], [], [], []
    for jb in range(NJB):
        sl = pl.ds(jb * LANES, LANES)
        mnb.append(pbuf[0, sl])
        deltab.append(pbuf[1, sl])
        invdb.append(pbuf[2, sl])
        halfwb.append(pbuf[3, sl])
        baseb.append((jnp.arange(LANES, dtype=jnp.int32) + jb * LANES) * GS)

    ones = jnp.ones((LANES,), jnp.float32)
    rcp.wait()
    fcp.wait()

    @plsc.parallel_loop(0, ROWS, unroll=4)
    def _(i):
        for jb in range(NJB):
            sl = pl.ds(jb * LANES, LANES)
            xr_v = rbuf[i, sl]
            tr = (xr_v - mnb[jb]) * invdb[jb]
            ir = jnp.minimum(tr.astype(jnp.int32), NB - 1)
            plsc.addupdate_scatter(cr, [baseb[jb] + ir], ones)
            xf_v = fbuf[i, sl]
            tf = (xf_v - mnb[jb]) * invdb[jb]
            tf = jnp.minimum(jnp.maximum(tf, -1.0), 64.0)
            jf = tf.astype(jnp.int32)
            jf = jnp.minimum(jnp.maximum(jf, 0), NB - 1)
            center = mnb[jb] + deltab[jb] * (jf.astype(jnp.float32) + 0.5)
            hit = halfwb[jb] > jnp.abs(xf_v - center)
            plsc.addupdate_scatter(cf, [baseb[jb] + jf], ones, mask=hit)

    pltpu.sync_copy(cr, counts_hbm.at[wid, 0])
    pltpu.sync_copy(cf, counts_hbm.at[wid, 1])


def _tc_finalize_body(c_ref, p_ref, o_ref):
    c = c_ref[...]
    tot = jnp.sum(c[:, :, :, :NB], axis=0)
    s = jnp.sum(jnp.abs(tot[1] - tot[0]), axis=1)
    delta = p_ref[1, :]
    o_ref[...] = s / (delta * float(NB * N))


_tc_finalize = pl.pallas_call(
    _tc_finalize_body,
    out_shape=jax.ShapeDtypeStruct((G,), jnp.float32),
)


def kernel(x_fake, x_real):
    xr = x_real.reshape(N, G)
    xf = x_fake.reshape(N, G)
    params = _tc_params(xr)
    counts = _sc_hist(xr, xf, params)
    losses = _tc_finalize(counts.reshape(NW, 2, G, GS), params)
    return losses.reshape(L, D)


# speedup vs baseline: 1.9371x; 1.8232x over previous
import functools

import jax
import jax.numpy as jnp
from jax import lax
from jax.experimental import pallas as pl
from jax.experimental.pallas import tpu as pltpu
from jax.experimental.pallas import tpu_sc as plsc

N = 16384
L = 16
D = 4
G = L * D
NB = 64
NC = 2
NS = 16
NW = NC * NS
GPT = G // NW
LANES = 16
GS = NB + 1
SUBH = LANES * GS
CWORDS = GPT * SUBH

_mesh = plsc.VectorSubcoreMesh(
    core_axis_name="c", subcore_axis_name="s", num_cores=NC, num_subcores=NS)
_params = pltpu.CompilerParams(
    needs_layout_passes=False, use_tc_tiling_on_sc=False)

_ACC = 4


@functools.partial(
    pl.kernel,
    out_type=jax.ShapeDtypeStruct((NW, LANES), jnp.float32),
    mesh=_mesh,
    compiler_params=_params,
    scratch_types=[pltpu.VMEM((GPT * N,), jnp.float32),
                   pltpu.VMEM((GPT * N,), jnp.float32),
                   pltpu.VMEM((CWORDS,), jnp.float32),
                   pltpu.VMEM((CWORDS,), jnp.float32),
                   pltpu.VMEM((LANES,), jnp.float32),
                   pltpu.SemaphoreType.DMA,
                   pltpu.SemaphoreType.DMA],
)
def _sc_hist_loss(xr_hbm, xf_hbm, out_hbm, rbuf, fbuf, cr, cf, obuf,
                  rsem, fsem):
    wid = lax.axis_index("s") * NC + lax.axis_index("c")
    e0 = wid * (GPT * N)
    rcp = pltpu.async_copy(xr_hbm.at[pl.ds(e0, GPT * N)], rbuf, rsem)
    fcp = pltpu.async_copy(xf_hbm.at[pl.ds(e0, GPT * N)], fbuf, fsem)

    zeros = jnp.zeros((LANES,), jnp.float32)

    @plsc.parallel_loop(0, CWORDS // LANES, unroll=5)
    def _(i):
        cr[pl.ds(i * LANES, LANES)] = zeros
        cf[pl.ds(i * LANES, LANES)] = zeros

    rcp.wait()
    fcp.wait()

    lane = jnp.arange(LANES, dtype=jnp.int32)
    ones = jnp.ones((LANES,), jnp.float32)
    losses = []
    for g in range(GPT):
        base = g * N
        first = [rbuf[pl.ds(base + k * LANES, LANES)] for k in range(_ACC)]

        def mbody(i, carry, base=base):
            out = []
            for k in range(_ACC):
                x = rbuf[pl.ds(base + (i * _ACC + k) * LANES, LANES)]
                out.append(jnp.minimum(carry[k], x))
            for k in range(_ACC):
                x = rbuf[pl.ds(base + (i * _ACC + k) * LANES, LANES)]
                out.append(jnp.maximum(carry[_ACC + k], x))
            return tuple(out)

        red = lax.fori_loop(1, N // LANES // _ACC, mbody, tuple(first) * 2)
        mn_v = jnp.minimum(jnp.minimum(red[0], red[1]),
                           jnp.minimum(red[2], red[3]))
        mx_v = jnp.maximum(jnp.maximum(red[4], red[5]),
                           jnp.maximum(red[6], red[7]))
        mn = jnp.min(mn_v)
        mx = jnp.max(mx_v)
        degen = jnp.abs(mx - mn) < 1e-10
        mx = jnp.where(degen, mx + 1e-05, mx)
        mn = jnp.where(degen, mn - 1e-05, mn)
        delta = (mx - mn) * (1.0 / NB)
        mnb = jnp.full((LANES,), mn, jnp.float32)
        deltab = jnp.full((LANES,), delta, jnp.float32)
        invdb = 1.0 / deltab
        halfwb = deltab * 0.5
        cbase = g * SUBH + lane * GS

        @plsc.parallel_loop(0, N // LANES, unroll=4)
        def _(i, base=base, mnb=mnb, deltab=deltab, invdb=invdb,
              halfwb=halfwb, cbase=cbase):
            xr_v = rbuf[pl.ds(base + i * LANES, LANES)]
            tr = (xr_v - mnb) * invdb
            ir = jnp.minimum(tr.astype(jnp.int32), NB - 1)
            plsc.addupdate_scatter(cr, [cbase + ir], ones)
            xf_v = fbuf[pl.ds(base + i * LANES, LANES)]
            tf = (xf_v - mnb) * invdb
            tf = jnp.minimum(jnp.maximum(tf, -1.0), 64.0)
            jf = tf.astype(jnp.int32)
            jf = jnp.minimum(jnp.maximum(jf, 0), NB - 1)
            center = mnb + deltab * (jf.astype(jnp.float32) + 0.5)
            hit = halfwb > jnp.abs(xf_v - center)
            plsc.addupdate_scatter(cf, [cbase + jf], ones, mask=hit)

        svec = jnp.zeros((LANES,), jnp.float32)
        for j in range(NB // LANES):
            ar = jnp.zeros((LANES,), jnp.float32)
            af = jnp.zeros((LANES,), jnp.float32)
            for s in range(LANES):
                off = g * SUBH + s * GS + j * LANES
                ar = ar + cr[pl.ds(off, LANES)]
                af = af + cf[pl.ds(off, LANES)]
            svec = svec + jnp.abs(af - ar)
        losses.append((jnp.sum(svec), delta))

    lv = jnp.where(lane == 0, losses[0][0], losses[1][0])
    dv = jnp.where(lane == 0, losses[0][1], losses[1][1])
    obuf[...] = lv / (dv * float(NB * N))
    pltpu.sync_copy(obuf, out_hbm.at[wid])


def kernel(x_fake, x_real):
    xr = jnp.transpose(x_real, (1, 2, 0)).reshape(G * N)
    xf = jnp.transpose(x_fake, (1, 2, 0)).reshape(G * N)
    out = _sc_hist_loss(xr, xf)
    return out[:, :GPT].reshape(L, D)


# speedup vs baseline: 1.9646x; 1.0142x over previous
import functools

import jax
import jax.numpy as jnp
from jax import lax
from jax.experimental import pallas as pl
from jax.experimental.pallas import tpu as pltpu
from jax.experimental.pallas import tpu_sc as plsc

N = 16384
L = 16
D = 4
G = L * D
NB = 64
NC = 2
NS = 16
NW = NC * NS
GPT = G // NW
LANES = 16
GS = NB + 1
SUBH = LANES * GS
CWORDS = GPT * SUBH

_mesh = plsc.VectorSubcoreMesh(
    core_axis_name="c", subcore_axis_name="s", num_cores=NC, num_subcores=NS)
_params = pltpu.CompilerParams(
    needs_layout_passes=False, use_tc_tiling_on_sc=False)

_ACC = 8


@functools.partial(
    pl.kernel,
    out_type=jax.ShapeDtypeStruct((NW, LANES), jnp.float32),
    mesh=_mesh,
    compiler_params=_params,
    scratch_types=[pltpu.VMEM((GPT * N,), jnp.float32),
                   pltpu.VMEM((GPT * N,), jnp.float32),
                   pltpu.VMEM((CWORDS,), jnp.float32),
                   pltpu.VMEM((CWORDS,), jnp.float32),
                   pltpu.VMEM((LANES,), jnp.float32),
                   pltpu.SemaphoreType.DMA,
                   pltpu.SemaphoreType.DMA],
)
def _sc_hist_loss(xr_hbm, xf_hbm, out_hbm, rbuf, fbuf, cr, cf, obuf,
                  rsem, fsem):
    wid = lax.axis_index("s") * NC + lax.axis_index("c")
    e0 = wid * (GPT * N)
    rcp = pltpu.async_copy(xr_hbm.at[pl.ds(e0, GPT * N)], rbuf, rsem)
    fcp = pltpu.async_copy(xf_hbm.at[pl.ds(e0, GPT * N)], fbuf, fsem)

    zeros = jnp.zeros((LANES,), jnp.float32)

    @plsc.parallel_loop(0, CWORDS // LANES, unroll=5)
    def _(i):
        cr[pl.ds(i * LANES, LANES)] = zeros
        cf[pl.ds(i * LANES, LANES)] = zeros

    rcp.wait()
    fcp.wait()

    lane = jnp.arange(LANES, dtype=jnp.int32)
    ones = jnp.ones((LANES,), jnp.float32)
    losses = []
    for g in range(GPT):
        base = g * N
        first = [rbuf[pl.ds(base + k * LANES, LANES)] for k in range(_ACC)]

        def mbody(i, carry, base=base):
            out_mn, out_mx = [], []
            for k in range(_ACC):
                x = rbuf[pl.ds(base + (i * _ACC + k) * LANES, LANES)]
                out_mn.append(jnp.minimum(carry[k], x))
                out_mx.append(jnp.maximum(carry[_ACC + k], x))
            return tuple(out_mn + out_mx)

        red = lax.fori_loop(1, N // LANES // _ACC, mbody, tuple(first) * 2)
        mn_v = red[0]
        mx_v = red[_ACC]
        for k in range(1, _ACC):
            mn_v = jnp.minimum(mn_v, red[k])
            mx_v = jnp.maximum(mx_v, red[_ACC + k])
        mn = jnp.min(mn_v)
        mx = jnp.max(mx_v)
        degen = jnp.abs(mx - mn) < 1e-10
        mx = jnp.where(degen, mx + 1e-05, mx)
        mn = jnp.where(degen, mn - 1e-05, mn)
        delta = (mx - mn) * (1.0 / NB)
        mnb = jnp.full((LANES,), mn, jnp.float32)
        deltab = jnp.full((LANES,), delta, jnp.float32)
        invdb = 1.0 / deltab
        halfwb = deltab * 0.5
        cbase = g * SUBH + lane * GS

        @plsc.parallel_loop(0, N // LANES, unroll=2)
        def _(i, base=base, mnb=mnb, deltab=deltab, invdb=invdb,
              halfwb=halfwb, cbase=cbase):
            xr_v = rbuf[pl.ds(base + i * LANES, LANES)]
            tr = (xr_v - mnb) * invdb
            ir = jnp.minimum(tr.astype(jnp.int32), NB - 1)
            plsc.addupdate_scatter(cr, [cbase + ir], ones)
            xf_v = fbuf[pl.ds(base + i * LANES, LANES)]
            tf = (xf_v - mnb) * invdb
            tf = jnp.minimum(jnp.maximum(tf, -1.0), 64.0)
            jf = tf.astype(jnp.int32)
            jf = jnp.minimum(jnp.maximum(jf, 0), NB - 1)
            center = mnb + deltab * (jf.astype(jnp.float32) + 0.5)
            hit = halfwb > jnp.abs(xf_v - center)
            plsc.addupdate_scatter(cf, [cbase + jf], ones, mask=hit)

        nj = NB // LANES

        def fbody(s, carry, g=g):
            out = []
            for j in range(nj):
                off = g * SUBH + s * GS + j * LANES
                out.append(carry[j] + cr[pl.ds(off, LANES)])
            for j in range(nj):
                off = g * SUBH + s * GS + j * LANES
                out.append(carry[nj + j] + cf[pl.ds(off, LANES)])
            return tuple(out)

        acc = lax.fori_loop(0, LANES, fbody,
                            (jnp.zeros((LANES,), jnp.float32),) * (2 * nj))
        svec = jnp.abs(acc[nj] - acc[0])
        for j in range(1, nj):
            svec = svec + jnp.abs(acc[nj + j] - acc[j])
        losses.append((jnp.sum(svec), delta))

    lv = jnp.where(lane == 0, losses[0][0], losses[1][0])
    dv = jnp.where(lane == 0, losses[0][1], losses[1][1])
    obuf[...] = lv / (dv * float(NB * N))
    pltpu.sync_copy(obuf, out_hbm.at[wid])


def kernel(x_fake, x_real):
    xr = jnp.transpose(x_real, (1, 2, 0)).reshape(G * N)
    xf = jnp.transpose(x_fake, (1, 2, 0)).reshape(G * N)
    out = _sc_hist_loss(xr, xf)
    return out[:, :GPT].reshape(L, D)


# speedup vs baseline: 2.2399x; 1.1401x over previous
import functools

import jax
import jax.numpy as jnp
from jax import lax
from jax.experimental import pallas as pl
from jax.experimental.pallas import tpu as pltpu
from jax.experimental.pallas import tpu_sc as plsc

N = 16384
L = 16
D = 4
G = L * D
NB = 64
NC = 2
NS = 16
NW = NC * NS
GPT = G // NW
LANES = 16
GS = NB + 1
SUBH = LANES * GS
CWORDS = GPT * SUBH

_mesh = plsc.VectorSubcoreMesh(
    core_axis_name="c", subcore_axis_name="s", num_cores=NC, num_subcores=NS)
_params = pltpu.CompilerParams(
    needs_layout_passes=False, use_tc_tiling_on_sc=False)

_ACC = 8


@functools.partial(
    pl.kernel,
    out_type=jax.ShapeDtypeStruct((NW, LANES), jnp.float32),
    mesh=_mesh,
    compiler_params=_params,
    scratch_types=[pltpu.VMEM((GPT, 128, 128), jnp.float32),
                   pltpu.VMEM((GPT, 128, 128), jnp.float32),
                   pltpu.VMEM((CWORDS,), jnp.float32),
                   pltpu.VMEM((CWORDS,), jnp.float32),
                   pltpu.VMEM((LANES,), jnp.float32),
                   pltpu.SemaphoreType.DMA,
                   pltpu.SemaphoreType.DMA],
)
def _sc_hist_loss(xr_hbm, xf_hbm, out_hbm, rbuf, fbuf, cr, cf, obuf,
                  rsem, fsem):
    wid = lax.axis_index("s") * NC + lax.axis_index("c")
    l0 = (GPT * wid) // D
    d0 = (GPT * wid) % D
    l1 = (GPT * wid + 1) // D
    d1 = (GPT * wid + 1) % D
    rcp0 = pltpu.async_copy(xr_hbm.at[l0, :, d0, :], rbuf.at[0], rsem)
    fcp0 = pltpu.async_copy(xf_hbm.at[l0, :, d0, :], fbuf.at[0], fsem)
    rcp1 = pltpu.async_copy(xr_hbm.at[l1, :, d1, :], rbuf.at[1], rsem)
    fcp1 = pltpu.async_copy(xf_hbm.at[l1, :, d1, :], fbuf.at[1], fsem)

    zeros = jnp.zeros((LANES,), jnp.float32)

    @plsc.parallel_loop(0, CWORDS // LANES, unroll=5)
    def _(i):
        cr[pl.ds(i * LANES, LANES)] = zeros
        cf[pl.ds(i * LANES, LANES)] = zeros

    rcp0.wait()
    fcp0.wait()
    rcp1.wait()
    fcp1.wait()

    lane = jnp.arange(LANES, dtype=jnp.int32)
    ones = jnp.ones((LANES,), jnp.float32)
    losses = []
    for g in range(GPT):
        base = g * N
        first = [rbuf[g, 0, pl.ds(k * LANES, LANES)] for k in range(_ACC)]

        def mbody(i, carry, g=g):
            out_mn, out_mx = [], []
            for k in range(_ACC):
                j = i * _ACC + k
                x = rbuf[g, j // 8, pl.ds((j % 8) * LANES, LANES)]
                out_mn.append(jnp.minimum(carry[k], x))
                out_mx.append(jnp.maximum(carry[_ACC + k], x))
            return tuple(out_mn + out_mx)

        red = lax.fori_loop(1, N // LANES // _ACC, mbody, tuple(first) * 2)
        mn_v = red[0]
        mx_v = red[_ACC]
        for k in range(1, _ACC):
            mn_v = jnp.minimum(mn_v, red[k])
            mx_v = jnp.maximum(mx_v, red[_ACC + k])
        mn = jnp.min(mn_v)
        mx = jnp.max(mx_v)
        degen = jnp.abs(mx - mn) < 1e-10
        mx = jnp.where(degen, mx + 1e-05, mx)
        mn = jnp.where(degen, mn - 1e-05, mn)
        delta = (mx - mn) * (1.0 / NB)
        mnb = jnp.full((LANES,), mn, jnp.float32)
        deltab = jnp.full((LANES,), delta, jnp.float32)
        invdb = 1.0 / deltab
        halfwb = deltab * 0.5
        cbase = g * SUBH + lane * GS

        @plsc.parallel_loop(0, N // LANES, unroll=4)
        def _(i, g=g, mnb=mnb, deltab=deltab, invdb=invdb,
              halfwb=halfwb, cbase=cbase):
            xr_v = rbuf[g, i // 8, pl.ds((i % 8) * LANES, LANES)]
            tr = (xr_v - mnb) * invdb
            ir = jnp.minimum(tr.astype(jnp.int32), NB - 1)
            plsc.addupdate_scatter(cr, [cbase + ir], ones)
            xf_v = fbuf[g, i // 8, pl.ds((i % 8) * LANES, LANES)]
            tf = (xf_v - mnb) * invdb
            jf = tf.astype(jnp.int32)
            jf = jnp.minimum(jnp.maximum(jf, 0), NB - 1)
            center = mnb + deltab * (jf.astype(jnp.float32) + 0.5)
            hit = halfwb > jnp.abs(xf_v - center)
            plsc.addupdate_scatter(cf, [cbase + jf], ones, mask=hit)

        nj = NB // LANES

        def fbody(s, carry, g=g):
            out = []
            for j in range(nj):
                off = g * SUBH + s * GS + j * LANES
                out.append(carry[j] + cr[pl.ds(off, LANES)])
            for j in range(nj):
                off = g * SUBH + s * GS + j * LANES
                out.append(carry[nj + j] + cf[pl.ds(off, LANES)])
            return tuple(out)

        acc = lax.fori_loop(0, LANES, fbody,
                            (jnp.zeros((LANES,), jnp.float32),) * (2 * nj))
        svec = jnp.abs(acc[nj] - acc[0])
        for j in range(1, nj):
            svec = svec + jnp.abs(acc[nj + j] - acc[j])
        losses.append((jnp.sum(svec), delta))

    lv = jnp.where(lane == 0, losses[0][0], losses[1][0])
    dv = jnp.where(lane == 0, losses[0][1], losses[1][1])
    obuf[...] = lv / (dv * float(NB * N))
    pltpu.sync_copy(obuf, out_hbm.at[wid])


def _as_tiled_view(x):
    return x.transpose(1, 0, 2).reshape(L, 128, 128, D).transpose(0, 1, 3, 2)


def kernel(x_fake, x_real):
    out = _sc_hist_loss(_as_tiled_view(x_real), _as_tiled_view(x_fake))
    return out[:, :GPT].reshape(L, D)


# speedup vs baseline: 2.2494x; 1.0043x over previous
import functools

import jax
import jax.numpy as jnp
from jax import lax
from jax.experimental import pallas as pl
from jax.experimental.pallas import tpu as pltpu
from jax.experimental.pallas import tpu_sc as plsc

N = 16384
L = 16
D = 4
G = L * D
NB = 64
NC = 2
NS = 16
NW = NC * NS
GPT = G // NW
LANES = 16
GS = NB + 1
SUBH = LANES * GS
CWORDS = GPT * SUBH

_mesh = plsc.VectorSubcoreMesh(
    core_axis_name="c", subcore_axis_name="s", num_cores=NC, num_subcores=NS)
_params = pltpu.CompilerParams(
    needs_layout_passes=False, use_tc_tiling_on_sc=False)

_ACC = 8


@functools.partial(
    pl.kernel,
    out_type=jax.ShapeDtypeStruct((NW, LANES), jnp.float32),
    mesh=_mesh,
    compiler_params=_params,
    scratch_types=[pltpu.VMEM((GPT, 128, 128), jnp.float32),
                   pltpu.VMEM((GPT, 128, 128), jnp.float32),
                   pltpu.VMEM((CWORDS,), jnp.float32),
                   pltpu.VMEM((CWORDS,), jnp.float32),
                   pltpu.VMEM((LANES,), jnp.float32),
                   pltpu.SemaphoreType.DMA,
                   pltpu.SemaphoreType.DMA,
                   pltpu.SemaphoreType.DMA,
                   pltpu.SemaphoreType.DMA],
)
def _sc_hist_loss(xr_hbm, xf_hbm, out_hbm, rbuf, fbuf, cr, cf, obuf,
                  rsem0, fsem0, rsem1, fsem1):
    wid = lax.axis_index("s") * NC + lax.axis_index("c")
    l0 = (GPT * wid) // D
    d0 = (GPT * wid) % D
    l1 = (GPT * wid + 1) // D
    d1 = (GPT * wid + 1) % D
    rcp0 = pltpu.async_copy(xr_hbm.at[l0, :, d0, :], rbuf.at[0], rsem0)
    rcp1 = pltpu.async_copy(xr_hbm.at[l1, :, d1, :], rbuf.at[1], rsem1)
    fcp0 = pltpu.async_copy(xf_hbm.at[l0, :, d0, :], fbuf.at[0], fsem0)
    fcp1 = pltpu.async_copy(xf_hbm.at[l1, :, d1, :], fbuf.at[1], fsem1)

    zeros = jnp.zeros((LANES,), jnp.float32)

    @plsc.parallel_loop(0, CWORDS // LANES, unroll=5)
    def _(i):
        cr[pl.ds(i * LANES, LANES)] = zeros
        cf[pl.ds(i * LANES, LANES)] = zeros

    lane = jnp.arange(LANES, dtype=jnp.int32)
    ones = jnp.ones((LANES,), jnp.float32)
    rwaits = [rcp0.wait, rcp1.wait]
    fwaits = [fcp0.wait, fcp1.wait]
    params = []
    for g in range(GPT):
        rwaits[g]()
        first = [rbuf[g, 0, pl.ds(k * LANES, LANES)] for k in range(_ACC)]

        def mbody(i, carry, g=g):
            out_mn, out_mx = [], []
            for k in range(_ACC):
                j = i * _ACC + k
                x = rbuf[g, j // 8, pl.ds((j % 8) * LANES, LANES)]
                out_mn.append(jnp.minimum(carry[k], x))
                out_mx.append(jnp.maximum(carry[_ACC + k], x))
            return tuple(out_mn + out_mx)

        red = lax.fori_loop(1, N // LANES // _ACC, mbody, tuple(first) * 2)
        mn_v = red[0]
        mx_v = red[_ACC]
        for k in range(1, _ACC):
            mn_v = jnp.minimum(mn_v, red[k])
            mx_v = jnp.maximum(mx_v, red[_ACC + k])
        mn = jnp.min(mn_v)
        mx = jnp.max(mx_v)
        degen = jnp.abs(mx - mn) < 1e-10
        mx = jnp.where(degen, mx + 1e-05, mx)
        mn = jnp.where(degen, mn - 1e-05, mn)
        delta = (mx - mn) * (1.0 / NB)
        mnb = jnp.full((LANES,), mn, jnp.float32)
        deltab = jnp.full((LANES,), delta, jnp.float32)
        invdb = 1.0 / deltab
        halfwb = deltab * 0.5
        params.append((mnb, deltab, invdb, halfwb, delta))

    losses = []
    for g in range(GPT):
        mnb, deltab, invdb, halfwb, delta = params[g]
        cbase = g * SUBH + lane * GS
        fwaits[g]()

        @plsc.parallel_loop(0, N // LANES, unroll=4)
        def _(i, g=g, mnb=mnb, deltab=deltab, invdb=invdb,
              halfwb=halfwb, cbase=cbase):
            xr_v = rbuf[g, i // 8, pl.ds((i % 8) * LANES, LANES)]
            tr = (xr_v - mnb) * invdb
            ir = jnp.minimum(tr.astype(jnp.int32), NB - 1)
            plsc.addupdate_scatter(cr, [cbase + ir], ones)
            xf_v = fbuf[g, i // 8, pl.ds((i % 8) * LANES, LANES)]
            tf = (xf_v - mnb) * invdb
            jf = tf.astype(jnp.int32)
            jf = jnp.minimum(jnp.maximum(jf, 0), NB - 1)
            center = mnb + deltab * (jf.astype(jnp.float32) + 0.5)
            hit = halfwb > jnp.abs(xf_v - center)
            plsc.addupdate_scatter(cf, [cbase + jf], ones, mask=hit)

        nj = NB // LANES

        def fbody(s, carry, g=g):
            out = []
            for j in range(nj):
                off = g * SUBH + s * GS + j * LANES
                out.append(carry[j] + cr[pl.ds(off, LANES)])
            for j in range(nj):
                off = g * SUBH + s * GS + j * LANES
                out.append(carry[nj + j] + cf[pl.ds(off, LANES)])
            return tuple(out)

        acc = lax.fori_loop(0, LANES, fbody,
                            (jnp.zeros((LANES,), jnp.float32),) * (2 * nj))
        svec = jnp.abs(acc[nj] - acc[0])
        for j in range(1, nj):
            svec = svec + jnp.abs(acc[nj + j] - acc[j])
        losses.append((jnp.sum(svec), delta))

    lv = jnp.where(lane == 0, losses[0][0], losses[1][0])
    dv = jnp.where(lane == 0, losses[0][1], losses[1][1])
    obuf[...] = lv / (dv * float(NB * N))
    pltpu.sync_copy(obuf, out_hbm.at[wid])


def _as_tiled_view(x):
    return x.transpose(1, 0, 2).reshape(L, 128, 128, D).transpose(0, 1, 3, 2)


def kernel(x_fake, x_real):
    out = _sc_hist_loss(_as_tiled_view(x_real), _as_tiled_view(x_fake))
    return out[:, :GPT].reshape(L, D)
